# baseline (speedup 1.0000x reference)
import functools

import jax
import jax.numpy as jnp
from jax import lax
from jax.experimental import pallas as pl
from jax.experimental.pallas import tpu as pltpu
from jax.experimental.pallas import tpu_sc as plsc

N = 10000
E = 320000
D = 128
G = 64
NC = 2
NS = 16
NW = NC * NS
N_PAD = 10240
GP = 128

_MESH = dict(core_axis_name="c", subcore_axis_name="s",
             num_cores=NC, num_subcores=NS)

EPT = E // NW
ECH = 96
CPG = 1
NPAR = 3
LOOK = NPAR - 1
EPT_PAD = 10368
NCH = EPT_PAD // ECH
NG = NCH // CPG
NSLOT = 4
GSUP = 12
ZR = 8


def _fill_zeros(zbuf, nrows):
    zv = jnp.zeros((16,), jnp.float32)
    for i in range(nrows):
        for j in range(D // 16):
            zbuf[i, pl.ds(j * 16, 16)] = zv


def _make_agg():
    mesh = plsc.VectorSubcoreMesh(**_MESH)
    scratch = [
        pltpu.VMEM_SHARED((N_PAD, D), jnp.float32),
        pltpu.VMEM((NPAR, CPG, ECH, D), jnp.float32),
        pltpu.VMEM((NSLOT, 2 * CPG, ECH), jnp.int32),
        pltpu.VMEM((ZR, D), jnp.float32),
    ] + [pltpu.SemaphoreType.DMA] * (NPAR * CPG + NPAR + NSLOT + 1)

    @functools.partial(
        pl.kernel,
        out_type=jax.ShapeDtypeStruct((NC, N_PAD, D), jnp.float32),
        mesh=mesh, scratch_types=scratch)
    def agg(h_hbm, idx_hbm, out_hbm, acc, rows, igrp, zbuf, *sems):
        sem_g = sems[:NPAR * CPG]
        sem_s = sems[NPAR * CPG:NPAR * CPG + NPAR]
        sem_il = sems[NPAR * CPG + NPAR:NPAR * CPG + NPAR + NSLOT]
        sem_z = sems[-1]
        c = lax.axis_index("c")
        s = lax.axis_index("s")
        wid = c * NS + s

        def load_idx(g, slot):
            pltpu.async_copy(idx_hbm.at[wid, g], igrp.at[slot], sem_il[slot])

        def wait_idx(slot):
            pltpu.make_async_copy(idx_hbm.at[wid, 0], igrp.at[slot],
                                  sem_il[slot]).wait()

        def issue_gathers(slot, q):
            for b in range(CPG):
                pltpu.async_copy(h_hbm.at[igrp.at[slot, 2 * b]],
                                 rows.at[q, b], sem_g[q * CPG + b])

        def wait_gather(q, b):
            pltpu.make_async_copy(h_hbm.at[igrp.at[0, 0]], rows.at[q, b],
                                  sem_g[q * CPG + b]).wait()

        def issue_scatters(slot, q):
            for b in range(CPG):
                pltpu.async_copy(rows.at[q, b],
                                 acc.at[igrp.at[slot, 2 * b + 1]],
                                 sem_s[q], add=True)

        def wait_scatters(q):
            for b in range(CPG):
                pltpu.make_async_copy(rows.at[q, b], acc.at[igrp.at[0, 1]],
                                      sem_s[q]).wait()

        for slot in range(NSLOT - 1):
            load_idx(slot, slot)

        _fill_zeros(zbuf, ZR)
        rpt = N_PAD // NS
        for i in range(rpt // ZR):
            pltpu.async_copy(zbuf, acc.at[pl.ds(s * rpt + i * ZR, ZR), :],
                             sem_z)
        for g0 in range(LOOK):
            wait_idx(g0)
            issue_gathers(g0, g0)
        for i in range(rpt // ZR):
            pltpu.make_async_copy(zbuf, acc.at[pl.ds(0, ZR), :], sem_z).wait()
        plsc.subcore_barrier()

        def outer(i, carry):
            for k in range(GSUP):
                g = GSUP * i + k
                p = k % NPAR
                for b in range(CPG):
                    wait_gather(p, b)
                issue_scatters(k % NSLOT, p)

                @pl.when(g + LOOK < NG)
                def _():
                    wait_idx((k + LOOK) % NSLOT)

                @pl.when(g >= 1)
                def _():
                    wait_scatters((p + LOOK) % NPAR)

                @pl.when(g + LOOK < NG)
                def _():
                    issue_gathers((k + LOOK) % NSLOT, (p + LOOK) % NPAR)

                @pl.when(g + NSLOT - 1 < NG)
                def _():
                    load_idx(g + NSLOT - 1, (k + NSLOT - 1) % NSLOT)
            return carry

        lax.fori_loop(0, NG // GSUP, outer, 0)
        wait_scatters((NG - 1) % NPAR)
        plsc.subcore_barrier()
        pltpu.sync_copy(acc.at[pl.ds(s * rpt, rpt), :],
                        out_hbm.at[c, pl.ds(s * rpt, rpt), :])

    return agg


def _make_pool():
    mesh = plsc.VectorSubcoreMesh(**_MESH)
    NPT = N_PAD // NW
    CHP = 80
    scratch = [
        pltpu.VMEM_SHARED((GP, D), jnp.float32),
        pltpu.VMEM((CHP, D), jnp.float32),
        pltpu.VMEM((CHP,), jnp.int32),
        pltpu.VMEM((GP // NS, D), jnp.float32),
        pltpu.SemaphoreType.DMA,
    ]

    @functools.partial(
        pl.kernel,
        out_type=jax.ShapeDtypeStruct((NC, GP, D), jnp.float32),
        mesh=mesh, scratch_types=scratch)
    def pool(h_hbm, b_hbm, out_hbm, acc, rows, bidx, zbuf, sem):
        c = lax.axis_index("c")
        s = lax.axis_index("s")
        rpt = GP // NS
        _fill_zeros(zbuf, rpt)
        pltpu.sync_copy(zbuf, acc.at[pl.ds(s * rpt, rpt), :])
        plsc.subcore_barrier()

        nbase = (c * NS + s) * NPT
        for j in range(NPT // CHP):
            b = nbase + j * CHP
            pltpu.sync_copy(b_hbm.at[pl.ds(b, CHP)], bidx)
            pltpu.sync_copy(h_hbm.at[pl.ds(b, CHP), :], rows)
            pltpu.sync_copy(rows, acc.at[bidx], add=True)
        plsc.subcore_barrier()
        pltpu.sync_copy(acc.at[pl.ds(s * rpt, rpt), :],
                        out_hbm.at[c, pl.ds(s * rpt, rpt), :])

    return pool


_agg = _make_agg()
_pool = _make_pool()


def _mm_body(h_ref, p0_ref, p1_ref, w_ref, b_ref, o_ref, *, relu):
    z = h_ref[...] + p0_ref[0] + p1_ref[0]
    y = jnp.dot(z, w_ref[...], preferred_element_type=jnp.float32) + b_ref[...]
    if relu:
        y = jnp.maximum(y, 0.0)
    o_ref[...] = y


def _mm(h, P, w, b, relu):
    BLK = 2048
    return pl.pallas_call(
        functools.partial(_mm_body, relu=relu),
        grid=(N_PAD // BLK,),
        in_specs=[
            pl.BlockSpec((BLK, D), lambda i: (i, 0)),
            pl.BlockSpec((1, BLK, D), lambda i: (0, i, 0)),
            pl.BlockSpec((1, BLK, D), lambda i: (1, i, 0)),
            pl.BlockSpec((D, D), lambda i: (0, 0)),
            pl.BlockSpec((1, D), lambda i: (0, 0)),
        ],
        out_specs=pl.BlockSpec((BLK, D), lambda i: (i, 0)),
        out_shape=jax.ShapeDtypeStruct((N_PAD, D), jnp.float32),
    )(h, P, P, w, b.reshape(1, D))


def _combine_body(p_ref, o_ref):
    o_ref[...] = p_ref[0, :G, :] + p_ref[1, :G, :]


def _combine(Ppool):
    return pl.pallas_call(
        _combine_body,
        out_shape=jax.ShapeDtypeStruct((G, D), jnp.float32),
    )(Ppool)


def kernel(x, edge_index, batch, W1, b1, W2, b2, W3, b3):
    trash = jnp.broadcast_to(
        N + jnp.arange(EPT_PAD - EPT, dtype=jnp.int32) % (N_PAD - N),
        (NW, EPT_PAD - EPT))
    src = jnp.concatenate([edge_index[0].reshape(NW, EPT), trash],
                          axis=1).reshape(NW, NG, CPG, ECH)
    dst = jnp.concatenate([edge_index[1].reshape(NW, EPT), trash],
                          axis=1).reshape(NW, NG, CPG, ECH)
    idx = jnp.stack([src, dst], axis=3).reshape(NW, NG, 2 * CPG, ECH)
    h = jnp.pad(x, ((0, N_PAD - N), (0, 0)))
    batch_p = jnp.pad(batch, (0, N_PAD - N), constant_values=G)

    P = _agg(h, idx)
    h = _mm(h, P, W1, b1, relu=True)
    P = _agg(h, idx)
    h = _mm(h, P, W2, b2, relu=True)
    P = _agg(h, idx)
    h = _mm(h, P, W3, b3, relu=False)
    Pp = _pool(h, batch_p)
    return _combine(Pp).reshape(-1)


# speedup vs baseline: 1.0175x; 1.0175x over previous
import functools

import jax
import jax.numpy as jnp
from jax import lax
from jax.experimental import pallas as pl
from jax.experimental.pallas import tpu as pltpu
from jax.experimental.pallas import tpu_sc as plsc

N = 10000
E = 320000
D = 128
G = 64
NC = 2
NS = 16
NW = NC * NS
N_PAD = 10240
GP = 128

_MESH = dict(core_axis_name="c", subcore_axis_name="s",
             num_cores=NC, num_subcores=NS)

EPT = E // NW
ECH = 96
CPG = 1
NPAR = 3
LOOK = NPAR - 1
EPT_PAD = 10368
NCH = EPT_PAD // ECH
NG = NCH // CPG
NSLOT = 4
GSUP = 12
ZR = 8


def _fill_zeros(zbuf, nrows):
    zv = jnp.zeros((16,), jnp.float32)
    for i in range(nrows):
        for j in range(D // 16):
            zbuf[i, pl.ds(j * 16, 16)] = zv


def _make_agg():
    mesh = plsc.VectorSubcoreMesh(**_MESH)
    scratch = [
        pltpu.VMEM_SHARED((N_PAD, D), jnp.float32),
        pltpu.VMEM((NPAR, CPG, ECH, D), jnp.float32),
        pltpu.VMEM((NSLOT, CPG, ECH), jnp.int32),
        pltpu.VMEM((NSLOT, CPG, ECH), jnp.int32),
        pltpu.VMEM((ZR, D), jnp.float32),
    ] + [pltpu.SemaphoreType.DMA] * (NPAR * CPG + NPAR + NSLOT + 1)

    @functools.partial(
        pl.kernel,
        out_type=jax.ShapeDtypeStruct((NC, N_PAD, D), jnp.float32),
        mesh=mesh, scratch_types=scratch)
    def agg(h_hbm, src_hbm, dst_hbm, out_hbm, acc, rows, sgrp, dgrp, zbuf,
            *sems):
        sem_g = sems[:NPAR * CPG]
        sem_s = sems[NPAR * CPG:NPAR * CPG + NPAR]
        sem_il = sems[NPAR * CPG + NPAR:NPAR * CPG + NPAR + NSLOT]
        sem_z = sems[-1]
        c = lax.axis_index("c")
        s = lax.axis_index("s")
        wid = c * NS + s

        def load_idx(g, slot):
            pltpu.async_copy(src_hbm.at[wid, g], sgrp.at[slot], sem_il[slot])
            pltpu.async_copy(dst_hbm.at[wid, g], dgrp.at[slot], sem_il[slot])

        def wait_idx(slot):
            pltpu.make_async_copy(src_hbm.at[wid, 0], sgrp.at[slot],
                                  sem_il[slot]).wait()
            pltpu.make_async_copy(dst_hbm.at[wid, 0], dgrp.at[slot],
                                  sem_il[slot]).wait()

        def issue_gathers(slot, q):
            for b in range(CPG):
                pltpu.async_copy(h_hbm.at[sgrp.at[slot, b]], rows.at[q, b],
                                 sem_g[q * CPG + b])

        def wait_gather(q, b):
            pltpu.make_async_copy(h_hbm.at[sgrp.at[0, 0]], rows.at[q, b],
                                  sem_g[q * CPG + b]).wait()

        def issue_scatters(slot, q):
            for b in range(CPG):
                pltpu.async_copy(rows.at[q, b], acc.at[dgrp.at[slot, b]],
                                 sem_s[q], add=True)

        def wait_scatters(q):
            for b in range(CPG):
                pltpu.make_async_copy(rows.at[q, b], acc.at[dgrp.at[0, 0]],
                                      sem_s[q]).wait()

        for slot in range(NSLOT - 1):
            load_idx(slot, slot)

        _fill_zeros(zbuf, ZR)
        rpt = N_PAD // NS
        for i in range(rpt // ZR):
            pltpu.async_copy(zbuf, acc.at[pl.ds(s * rpt + i * ZR, ZR), :],
                             sem_z)
        for g0 in range(LOOK):
            wait_idx(g0)
            issue_gathers(g0, g0)
        for i in range(rpt // ZR):
            pltpu.make_async_copy(zbuf, acc.at[pl.ds(0, ZR), :], sem_z).wait()
        plsc.subcore_barrier()

        def outer(i, carry):
            for k in range(GSUP):
                g = GSUP * i + k
                p = k % NPAR
                for b in range(CPG):
                    wait_gather(p, b)
                issue_scatters(k % NSLOT, p)

                @pl.when(g + LOOK < NG)
                def _():
                    wait_idx((k + LOOK) % NSLOT)

                @pl.when(g >= 1)
                def _():
                    wait_scatters((p + LOOK) % NPAR)

                @pl.when(g + LOOK < NG)
                def _():
                    issue_gathers((k + LOOK) % NSLOT, (p + LOOK) % NPAR)

                @pl.when(g + NSLOT - 1 < NG)
                def _():
                    load_idx(g + NSLOT - 1, (k + NSLOT - 1) % NSLOT)
            return carry

        lax.fori_loop(0, NG // GSUP, outer, 0)
        wait_scatters((NG - 1) % NPAR)
        plsc.subcore_barrier()
        pltpu.sync_copy(acc.at[pl.ds(s * rpt, rpt), :],
                        out_hbm.at[c, pl.ds(s * rpt, rpt), :])

    return agg


def _make_pool():
    mesh = plsc.VectorSubcoreMesh(**_MESH)
    NPT = N_PAD // NW
    CHP = 80
    scratch = [
        pltpu.VMEM_SHARED((GP, D), jnp.float32),
        pltpu.VMEM((CHP, D), jnp.float32),
        pltpu.VMEM((CHP,), jnp.int32),
        pltpu.VMEM((GP // NS, D), jnp.float32),
        pltpu.SemaphoreType.DMA,
    ]

    @functools.partial(
        pl.kernel,
        out_type=jax.ShapeDtypeStruct((NC, GP, D), jnp.float32),
        mesh=mesh, scratch_types=scratch)
    def pool(h_hbm, b_hbm, out_hbm, acc, rows, bidx, zbuf, sem):
        c = lax.axis_index("c")
        s = lax.axis_index("s")
        rpt = GP // NS
        _fill_zeros(zbuf, rpt)
        pltpu.sync_copy(zbuf, acc.at[pl.ds(s * rpt, rpt), :])
        plsc.subcore_barrier()

        nbase = (c * NS + s) * NPT
        for j in range(NPT // CHP):
            b = nbase + j * CHP
            pltpu.sync_copy(b_hbm.at[pl.ds(b, CHP)], bidx)
            pltpu.sync_copy(h_hbm.at[pl.ds(b, CHP), :], rows)
            pltpu.sync_copy(rows, acc.at[bidx], add=True)
        plsc.subcore_barrier()
        pltpu.sync_copy(acc.at[pl.ds(s * rpt, rpt), :],
                        out_hbm.at[c, pl.ds(s * rpt, rpt), :])

    return pool


_agg = _make_agg()
_pool = _make_pool()


def _mm_body(h_ref, p0_ref, p1_ref, w_ref, b_ref, o_ref, *, relu):
    z = h_ref[...] + p0_ref[0] + p1_ref[0]
    y = jnp.dot(z, w_ref[...], preferred_element_type=jnp.float32) + b_ref[...]
    if relu:
        y = jnp.maximum(y, 0.0)
    o_ref[...] = y


def _mm(h, P, w, b, relu):
    BLK = 2048
    return pl.pallas_call(
        functools.partial(_mm_body, relu=relu),
        grid=(N_PAD // BLK,),
        in_specs=[
            pl.BlockSpec((BLK, D), lambda i: (i, 0)),
            pl.BlockSpec((1, BLK, D), lambda i: (0, i, 0)),
            pl.BlockSpec((1, BLK, D), lambda i: (1, i, 0)),
            pl.BlockSpec((D, D), lambda i: (0, 0)),
            pl.BlockSpec((1, D), lambda i: (0, 0)),
        ],
        out_specs=pl.BlockSpec((BLK, D), lambda i: (i, 0)),
        out_shape=jax.ShapeDtypeStruct((N_PAD, D), jnp.float32),
    )(h, P, P, w, b.reshape(1, D))


def _combine_body(p_ref, o_ref):
    o_ref[...] = p_ref[0, :G, :] + p_ref[1, :G, :]


def _combine(Ppool):
    return pl.pallas_call(
        _combine_body,
        out_shape=jax.ShapeDtypeStruct((G, D), jnp.float32),
    )(Ppool)


def kernel(x, edge_index, batch, W1, b1, W2, b2, W3, b3):
    trash = jnp.broadcast_to(
        N + jnp.arange(EPT_PAD - EPT, dtype=jnp.int32) % (N_PAD - N),
        (NW, EPT_PAD - EPT))
    src = jnp.concatenate([edge_index[0].reshape(NW, EPT), trash],
                          axis=1).reshape(NW, NG, CPG, ECH)
    dst = jnp.concatenate([edge_index[1].reshape(NW, EPT), trash],
                          axis=1).reshape(NW, NG, CPG, ECH)
    h = jnp.pad(x, ((0, N_PAD - N), (0, 0)))
    batch_p = jnp.pad(batch, (0, N_PAD - N), constant_values=G)

    P = _agg(h, src, dst)
    h = _mm(h, P, W1, b1, relu=True)
    P = _agg(h, src, dst)
    h = _mm(h, P, W2, b2, relu=True)
    P = _agg(h, src, dst)
    h = _mm(h, P, W3, b3, relu=False)
    Pp = _pool(h, batch_p)
    return _combine(Pp).reshape(-1)
